# baseline (speedup 1.0000x reference)
import functools

import jax
import jax.numpy as jnp
from jax import lax
from jax.experimental import pallas as pl
from jax.experimental.pallas import tpu as pltpu
from jax.experimental.pallas import tpu_sc as plsc

B = 16384
D = 128
NCHUNK = 1
CB = B // NCHUNK
BLK = 4096
NBLK = CB // BLK
NBUF = 3


def _gather3_sc(user, item, time, user_table, item_table, time_table):
    info = plsc.get_sparse_core_info()
    nw = info.num_cores * info.num_subcores
    bpw = CB // nw
    chunk = min(bpw, 256)
    mesh = plsc.VectorSubcoreMesh(core_axis_name="c", subcore_axis_name="s")

    @functools.partial(
        pl.kernel,
        mesh=mesh,
        out_type=[
            jax.ShapeDtypeStruct((CB, D), jnp.float32),
            jax.ShapeDtypeStruct((CB, D), jnp.float32),
            jax.ShapeDtypeStruct((CB, D), jnp.float32),
        ],
        scratch_types=[
            pltpu.VMEM((3 * bpw,), jnp.int32),
        ] + [pltpu.VMEM((chunk, D), jnp.float32)] * NBUF
          + [pltpu.SemaphoreType.DMA] * NBUF,
    )
    def gather3(idx_all, u_tbl, i_tbl, t_tbl,
                u_out, i_out, t_out, idx_v, *rest):
        bufs = rest[:NBUF]
        sems = rest[NBUF:]
        wid = lax.axis_index("s") * info.num_cores + lax.axis_index("c")
        base = wid * bpw
        pltpu.sync_copy(idx_all.at[wid], idx_v)
        work = []
        for c in range(bpw // chunk):
            for f, (tbl, out) in enumerate(((u_tbl, u_out),
                                            (i_tbl, i_out),
                                            (t_tbl, t_out))):
                work.append((f, tbl, out, c * chunk))
        n = len(work)
        depth = 2
        g_copies = [None] * n
        w_copies = [None] * n

        def g_start(k):
            f, tbl, _, off = work[k]
            g_copies[k] = pltpu.async_copy(
                tbl.at[idx_v.at[pl.ds(f * bpw + off, chunk)]],
                bufs[k % NBUF], sems[k % NBUF])

        for k in range(min(depth, n)):
            g_start(k)
        for k in range(n):
            g_copies[k].wait()
            if k + depth < n:
                if k + depth >= NBUF:
                    w_copies[k + depth - NBUF].wait()
                g_start(k + depth)
            _, _, out, off = work[k]
            w_copies[k] = pltpu.async_copy(
                bufs[k % NBUF], out.at[pl.ds(base + off, chunk)],
                sems[k % NBUF])
        for k in range(max(0, n - NBUF), n):
            w_copies[k].wait()

    idx_all = jnp.stack(
        [user.reshape(nw, bpw), item.reshape(nw, bpw), time.reshape(nw, bpw)],
        axis=1).reshape(nw, 3 * bpw)
    return gather3(idx_all, user_table, item_table, time_table)


def _tc_body(u_ref, i_ref, t_ref, wu_ref, wi_ref, wt_ref, b_ref, o_ref):
    dn = (((1,), (1,)), ((), ()))
    u = lax.dot_general(wu_ref[...], u_ref[...], dn,
                        preferred_element_type=jnp.float32) + b_ref[:, 0:1]
    i = lax.dot_general(wi_ref[...], i_ref[...], dn,
                        preferred_element_type=jnp.float32) + b_ref[:, 1:2]
    t = lax.dot_general(wt_ref[...], t_ref[...], dn,
                        preferred_element_type=jnp.float32) + b_ref[:, 2:3]
    s = jnp.sum(u * i * t, axis=0)
    o_ref[...] = jax.nn.sigmoid(s)


def _compute_tc(u_rows, i_rows, t_rows, Wu, Wi, Wt, bias, interpret=False):
    blk_spec = pl.BlockSpec((BLK, D), lambda i: (i, 0))
    w_spec = pl.BlockSpec((D, D), lambda i: (0, 0))
    b_spec = pl.BlockSpec((D, 3), lambda i: (0, 0))
    out_spec = pl.BlockSpec((BLK,), lambda i: (i,))
    return pl.pallas_call(
        _tc_body,
        grid=(NBLK,),
        in_specs=[blk_spec, blk_spec, blk_spec, w_spec, w_spec, w_spec, b_spec],
        out_specs=out_spec,
        out_shape=jax.ShapeDtypeStruct((CB,), jnp.float32),
        interpret=interpret,
    )(u_rows, i_rows, t_rows, Wu, Wi, Wt, bias)


def kernel(user, item, time, user_table, item_table, time_table,
           Wu, bu, Wi, bi, Wt, bt):
    user = user.astype(jnp.int32)
    item = item.astype(jnp.int32)
    time = time.astype(jnp.int32)
    bias = jnp.stack([bu, bi, bt], axis=1)
    outs = []
    for c in range(NCHUNK):
        sl = slice(c * CB, (c + 1) * CB)
        u_rows, i_rows, t_rows = _gather3_sc(
            user[sl], item[sl], time[sl],
            user_table, item_table, time_table)
        outs.append(_compute_tc(u_rows, i_rows, t_rows, Wu, Wi, Wt, bias))
    return jnp.concatenate(outs, axis=0)


# speedup vs baseline: 1.0435x; 1.0435x over previous
import functools

import jax
import jax.numpy as jnp
from jax import lax
from jax.experimental import pallas as pl
from jax.experimental.pallas import tpu as pltpu
from jax.experimental.pallas import tpu_sc as plsc

B = 16384
D = 128
BLK = 4096
NBLK = B // BLK
NBUF = 7
DEPTH = 6


def _gather3_sc(user, item, time, user_table, item_table, time_table):
    info = plsc.get_sparse_core_info()
    nw = info.num_cores * info.num_subcores
    bpw = B // nw
    chunk = min(bpw, 128)
    mesh = plsc.VectorSubcoreMesh(core_axis_name="c", subcore_axis_name="s")

    @functools.partial(
        pl.kernel,
        mesh=mesh,
        out_type=[
            jax.ShapeDtypeStruct((B, D), jnp.float32),
            jax.ShapeDtypeStruct((B, D), jnp.float32),
            jax.ShapeDtypeStruct((B, D), jnp.float32),
        ],
        scratch_types=[
            pltpu.VMEM((3 * bpw,), jnp.int32),
        ] + [pltpu.VMEM((chunk, D), jnp.float32)] * NBUF
          + [pltpu.SemaphoreType.DMA] * NBUF,
    )
    def gather3(idx_all, u_tbl, i_tbl, t_tbl,
                u_out, i_out, t_out, idx_v, *rest):
        bufs = rest[:NBUF]
        sems = rest[NBUF:]
        wid = lax.axis_index("s") * info.num_cores + lax.axis_index("c")
        base = wid * bpw
        pltpu.sync_copy(idx_all.at[wid], idx_v)
        work = []
        for c in range(bpw // chunk):
            for f, (tbl, out) in enumerate(((u_tbl, u_out),
                                            (i_tbl, i_out),
                                            (t_tbl, t_out))):
                work.append((f, tbl, out, c * chunk))
        n = len(work)
        g_copies = [None] * n
        w_copies = [None] * n

        def g_start(k):
            f, tbl, _, off = work[k]
            g_copies[k] = pltpu.async_copy(
                tbl.at[idx_v.at[pl.ds(f * bpw + off, chunk)]],
                bufs[k % NBUF], sems[k % NBUF])

        for k in range(min(DEPTH, n)):
            g_start(k)
        for k in range(n):
            g_copies[k].wait()
            if k + DEPTH < n:
                if k + DEPTH >= NBUF:
                    w_copies[k + DEPTH - NBUF].wait()
                g_start(k + DEPTH)
            _, _, out, off = work[k]
            w_copies[k] = pltpu.async_copy(
                bufs[k % NBUF], out.at[pl.ds(base + off, chunk)],
                sems[k % NBUF])
        for k in range(max(0, n - NBUF), n):
            w_copies[k].wait()

    idx_all = jnp.stack(
        [user.reshape(nw, bpw), item.reshape(nw, bpw), time.reshape(nw, bpw)],
        axis=1).reshape(nw, 3 * bpw)
    return gather3(idx_all, user_table, item_table, time_table)


def _tc_body(u_ref, i_ref, t_ref, wu_ref, wi_ref, wt_ref, b_ref, o_ref):
    dn = (((1,), (1,)), ((), ()))
    u = lax.dot_general(wu_ref[...], u_ref[...], dn,
                        preferred_element_type=jnp.float32) + b_ref[:, 0:1]
    i = lax.dot_general(wi_ref[...], i_ref[...], dn,
                        preferred_element_type=jnp.float32) + b_ref[:, 1:2]
    t = lax.dot_general(wt_ref[...], t_ref[...], dn,
                        preferred_element_type=jnp.float32) + b_ref[:, 2:3]
    s = jnp.sum(u * i * t, axis=0)
    o_ref[...] = jax.nn.sigmoid(s)


def _compute_tc(u_rows, i_rows, t_rows, Wu, Wi, Wt, bias, interpret=False):
    blk_spec = pl.BlockSpec((BLK, D), lambda i: (i, 0))
    w_spec = pl.BlockSpec((D, D), lambda i: (0, 0))
    b_spec = pl.BlockSpec((D, 3), lambda i: (0, 0))
    out_spec = pl.BlockSpec((BLK,), lambda i: (i,))
    return pl.pallas_call(
        _tc_body,
        grid=(NBLK,),
        in_specs=[blk_spec, blk_spec, blk_spec, w_spec, w_spec, w_spec, b_spec],
        out_specs=out_spec,
        out_shape=jax.ShapeDtypeStruct((B,), jnp.float32),
        interpret=interpret,
    )(u_rows, i_rows, t_rows, Wu, Wi, Wt, bias)


def kernel(user, item, time, user_table, item_table, time_table,
           Wu, bu, Wi, bi, Wt, bt):
    user = user.astype(jnp.int32)
    item = item.astype(jnp.int32)
    time = time.astype(jnp.int32)
    u_rows, i_rows, t_rows = _gather3_sc(
        user, item, time, user_table, item_table, time_table)
    bias = jnp.stack([bu, bi, bt], axis=1)
    return _compute_tc(u_rows, i_rows, t_rows, Wu, Wi, Wt, bias)
